# baseline (speedup 1.0000x reference)
import jax
import jax.numpy as jnp
from jax.experimental import pallas as pl


def _conv1_body(y_ref, yh_ref, w_ref, b_ref, o_ref):
    y = y_ref[0]
    last = jnp.equal(pl.program_id(1), 1)
    hrow = jnp.where(last, jnp.zeros((1, 128, 24), jnp.bfloat16),
                     yh_ref[0, 0:1])
    ys = y[..., 12:24]
    hs = hrow[..., 12:24]
    prev = jnp.pad(ys, ((0, 0), (1, 0), (0, 0)))[:, :128, :]
    hprev = jnp.pad(hs, ((0, 0), (1, 0), (0, 0)))[:, :128, :]
    prevn = jnp.concatenate([prev[1:], hprev], axis=0)
    yn = jnp.concatenate([y[1:], hrow], axis=0)
    x = jnp.concatenate([prev, y, prevn, yn], axis=-1)
    a = x.reshape(128 * 128, 72)
    out = jnp.dot(a, w_ref[...], preferred_element_type=jnp.float32)
    out = jax.nn.relu(out + b_ref[...])
    o_ref[0] = out.astype(jnp.bfloat16).reshape(128, 128, 128)


def _conv1(y24, w72, b):
    B = y24.shape[0]
    return pl.pallas_call(
        _conv1_body,
        grid=(B, 2),
        in_specs=[
            pl.BlockSpec((1, 128, 128, 24), lambda i, r: (i, r, 0, 0)),
            pl.BlockSpec((1, 8, 128, 24),
                         lambda i, r: (i, jnp.minimum(16 * r + 16, 31), 0, 0)),
            pl.BlockSpec((72, 128), lambda i, r: (0, 0)),
            pl.BlockSpec((1, 128), lambda i, r: (0, 0)),
        ],
        out_specs=pl.BlockSpec((1, 128, 128, 128), lambda i, r: (i, r, 0, 0)),
        out_shape=jax.ShapeDtypeStruct((B, 256, 128, 128), jnp.bfloat16),
    )(y24, y24, w72, b)


def _conv2_body(x_ref, w1_ref, w2_ref, b_ref, o_ref):
    x1c = x_ref[0].reshape(128, 2, 128, 128)
    zrow = jnp.zeros((1, 128, 128), jnp.bfloat16)
    even = x1c[:, 0]
    odd = x1c[:, 1]
    slabs = (
        jnp.concatenate([zrow, odd[0:127]], axis=0),
        even,
        odd,
    )
    acc = jnp.zeros((128 * 128, 64), dtype=jnp.float32)
    for dy in range(3):
        s = slabs[dy].reshape(128 * 128, 128)
        acc = acc + jnp.dot(s, w2_ref[dy], preferred_element_type=jnp.float32)
        t = jnp.dot(s, w1_ref[dy], preferred_element_type=jnp.float32)
        t = t.reshape(128, 128, 64)
        t = jnp.pad(t[:, :127, :], ((0, 0), (1, 0), (0, 0)))
        acc = acc + t.reshape(128 * 128, 64)
    y = jax.nn.relu(acc + b_ref[...])
    y = y.reshape(16, 8, 16, 8, 64)
    y = jnp.mean(y, axis=1)
    y = jnp.mean(y, axis=2)
    o_ref[0] = y


def _conv2_pool(x1c, wd1, wd2, b):
    B = x1c.shape[0]
    return pl.pallas_call(
        _conv2_body,
        grid=(B,),
        in_specs=[
            pl.BlockSpec((1, 256, 128, 128), lambda i: (i, 0, 0, 0)),
            pl.BlockSpec((3, 128, 64), lambda i: (0, 0, 0)),
            pl.BlockSpec((3, 128, 64), lambda i: (0, 0, 0)),
            pl.BlockSpec((1, 64), lambda i: (0, 0)),
        ],
        out_specs=pl.BlockSpec((1, 16, 16, 64), lambda i: (i, 0, 0, 0)),
        out_shape=jax.ShapeDtypeStruct((B, 16, 16, 64), jnp.float32),
    )(x1c, wd1, wd2, b)


def _mlp_body(pf_ref, bw1_ref, bb1_ref, bw2_ref, bb2_ref, bw3_ref, bb3_ref,
              sw_ref, sb_ref, big_ref, small_ref):
    f32 = jnp.float32
    pf = pf_ref[0].reshape(256, 3072).astype(jnp.bfloat16)
    big = jax.nn.relu(jnp.dot(pf, bw1_ref[...], preferred_element_type=f32)
                      + bb1_ref[...])
    big = jax.nn.relu(jnp.dot(big.astype(jnp.bfloat16), bw2_ref[...],
                              preferred_element_type=f32) + bb2_ref[...])
    big_ref[0] = (jnp.dot(big.astype(jnp.bfloat16), bw3_ref[...],
                          preferred_element_type=f32) + bb3_ref[...])
    small_ref[0] = (jnp.dot(pf, sw_ref[...], preferred_element_type=f32)
                    + sb_ref[...])


def _expert_mlps(pf, bw1, bb1, bw2, bb2, bw3, bb3, sw, sb):
    B = pf.shape[0]

    def rep(shape):
        nd = len(shape)
        return pl.BlockSpec(shape, lambda i, _n=nd: (0,) * _n)

    return pl.pallas_call(
        _mlp_body,
        grid=(B,),
        in_specs=[
            pl.BlockSpec((1, 256, 3, 32, 32), lambda i: (i, 0, 0, 0, 0)),
            rep((3072, 512)), rep((1, 512)), rep((512, 512)), rep((1, 512)),
            rep((512, 128)), rep((1, 128)),
            rep((3072, 128)), rep((1, 128)),
        ],
        out_specs=[
            pl.BlockSpec((1, 256, 128), lambda i: (i, 0, 0)),
            pl.BlockSpec((1, 256, 128), lambda i: (i, 0, 0)),
        ],
        out_shape=[
            jax.ShapeDtypeStruct((B, 256, 128), jnp.float32),
            jax.ShapeDtypeStruct((B, 256, 128), jnp.float32),
        ],
    )(pf, bw1, bb1, bw2, bb2, bw3, bb3, sw, sb)


def _head_body(feat_ref, big_ref, small_ref, aw1_ref, ab1_ref, aw2_ref,
               ab2_ref, thr_ref, gw_ref, gb_ref, dw1_ref, db1_ref,
               cw_ref, cb_ref, rw_ref, rb_ref, cls_ref, reg_ref):
    f32 = jnp.float32
    feat = feat_ref[...].reshape(8 * 256, 64)
    big = big_ref[...]
    small = small_ref[...]

    h = jax.nn.relu(jnp.dot(feat, aw1_ref[...], preferred_element_type=f32)
                    + ab1_ref[...])
    s = jax.nn.sigmoid(jnp.dot(h, aw2_ref[...], preferred_element_type=f32)
                       + ab2_ref[...])
    s3 = s.reshape(8, 256, 1)
    mask = jax.nn.sigmoid(s3 - thr_ref[0, 0])
    combined = mask * big + (1.0 - mask) * small

    m = jnp.max(s3, axis=1, keepdims=True)
    e = jnp.exp(s3 - m)
    attn = e / jnp.sum(e, axis=1, keepdims=True)
    weighted = jnp.sum(combined * attn, axis=1)
    mean_pool = jnp.mean(combined, axis=1)
    gw = gw_ref[...]
    gfeat = (jnp.dot(weighted, gw[:128], preferred_element_type=f32)
             + jnp.dot(mean_pool, gw[128:], preferred_element_type=f32)
             + gb_ref[...])

    dw1 = dw1_ref[...]
    hc = jnp.dot(combined.reshape(8 * 256, 128), dw1[:128],
                 preferred_element_type=f32).reshape(8, 256, 256)
    hg = jnp.dot(gfeat, dw1[128:], preferred_element_type=f32)
    hd = jax.nn.relu(hc + hg[:, None, :] + db1_ref[...])
    hd2 = hd.reshape(8 * 256, 256)
    cls = jnp.dot(hd2, cw_ref[...], preferred_element_type=f32) + cb_ref[...]
    reg = jnp.dot(hd2, rw_ref[...], preferred_element_type=f32) + rb_ref[...]
    cls_ref[...] = cls.reshape(8, 256, 80)
    reg_ref[...] = reg.reshape(8, 256, 4)


def _head(feat, big, small, aw1, ab1, aw2, ab2, thr, gw, gb, dw1, db1,
          cw, cb, rw, rb):
    return pl.pallas_call(
        _head_body,
        out_shape=[
            jax.ShapeDtypeStruct((8, 256, 80), jnp.float32),
            jax.ShapeDtypeStruct((8, 256, 4), jnp.float32),
        ],
    )(feat, big, small, aw1, ab1, aw2, ab2, thr, gw, gb, dw1, db1,
      cw, cb, rw, rb)


def _conv1_w72(conv1_w):
    w = jnp.zeros((72, 128), jnp.float32)
    for A in range(2):
        for CB in range(-1, 2):
            for dr in range(2):
                for dc in range(2):
                    ky = 2 * A + dr - 1
                    if not 0 <= ky <= 2:
                        continue
                    base = A * 36 + (CB + 1) * 12 + (dr * 2 + dc) * 3
                    kx_e = 2 * CB + dc + 1
                    if 0 <= kx_e <= 2:
                        w = w.at[base:base + 3, 0:64].set(
                            conv1_w[:, :, ky, kx_e].T)
                    kx_o = 2 * CB + dc - 1
                    if 0 <= kx_o <= 2:
                        w = w.at[base:base + 3, 64:128].set(
                            conv1_w[:, :, ky, kx_o].T)
    return w


def _conv2_w(conv2_w):
    wt = conv2_w.transpose(2, 3, 1, 0)
    z = jnp.zeros((64, 64), jnp.float32)
    wd1 = jnp.stack([jnp.concatenate([z, wt[dy, 0]], axis=0)
                     for dy in range(3)])
    wd2 = jnp.stack([jnp.concatenate([wt[dy, 1], wt[dy, 2]], axis=0)
                     for dy in range(3)])
    return wd1, wd2


def kernel(images, patches, conv1_w, conv1_b, conv2_w, conv2_b, attn_w1,
           attn_b1, attn_w2, attn_b2, threshold, big_w1, big_b1, big_w2,
           big_b2, big_w3, big_b3, small_w, small_b, agg_w, agg_b, det_w1,
           det_b1, det_cls_w, det_cls_b, det_reg_w, det_reg_b):
    B = images.shape[0]
    N = 256

    y24 = images.astype(jnp.bfloat16).reshape(B, 3, 256, 2, 128, 2, 2)
    y24 = y24.transpose(0, 2, 4, 5, 3, 6, 1).reshape(B, 256, 128, 24)

    w72 = _conv1_w72(conv1_w).astype(jnp.bfloat16)
    b1 = jnp.concatenate([conv1_b, conv1_b]).reshape(1, 128)
    x1c = _conv1(y24, w72, b1)

    wd1, wd2 = _conv2_w(conv2_w)
    feat = _conv2_pool(x1c, wd1.astype(jnp.bfloat16), wd2.astype(jnp.bfloat16),
                       conv2_b.reshape(1, 64))
    feat = feat.reshape(B, N, 64)

    bf16 = jnp.bfloat16
    big, small = _expert_mlps(
        patches,
        big_w1.astype(bf16), big_b1.reshape(1, 512),
        big_w2.astype(bf16), big_b2.reshape(1, 512),
        big_w3.astype(bf16), big_b3.reshape(1, 128),
        small_w.astype(bf16), small_b.reshape(1, 128))

    cls_logits, reg_preds = _head(
        feat, big, small,
        attn_w1, attn_b1.reshape(1, 32), attn_w2, attn_b2.reshape(1, 1),
        threshold.reshape(1, 1),
        agg_w, agg_b.reshape(1, 256),
        det_w1, det_b1.reshape(1, 256),
        det_cls_w, det_cls_b.reshape(1, 80),
        det_reg_w, det_reg_b.reshape(1, 4))
    return (cls_logits, reg_preds)


# speedup vs baseline: 1.3462x; 1.3462x over previous
import jax
import jax.numpy as jnp
from jax.experimental import pallas as pl


def _conv1_body(y_ref, yh_ref, w_ref, b_ref, o_ref):
    y = y_ref[0]
    last = jnp.equal(pl.program_id(1), 1)
    hrow = jnp.where(last, jnp.zeros((1, 128, 24), jnp.bfloat16),
                     yh_ref[0, 0:1])
    ys = y[..., 12:24]
    hs = hrow[..., 12:24]
    prev = jnp.pad(ys, ((0, 0), (1, 0), (0, 0)))[:, :128, :]
    hprev = jnp.pad(hs, ((0, 0), (1, 0), (0, 0)))[:, :128, :]
    x36 = jnp.concatenate([prev, y], axis=-1)
    h36 = jnp.concatenate([hprev, hrow], axis=-1)
    xb = jnp.concatenate([x36[1:], h36], axis=0)
    a = x36.reshape(128 * 128, 36)
    an = xb.reshape(128 * 128, 36)
    out = (jnp.dot(a, w_ref[0:36], preferred_element_type=jnp.float32)
           + jnp.dot(an, w_ref[36:72], preferred_element_type=jnp.float32))
    out = jax.nn.relu(out + b_ref[...])
    o_ref[0] = out.astype(jnp.bfloat16).reshape(128, 128, 128)


def _conv1(y24, w72, b):
    B = y24.shape[0]
    return pl.pallas_call(
        _conv1_body,
        grid=(B, 2),
        in_specs=[
            pl.BlockSpec((1, 128, 128, 24), lambda i, r: (i, r, 0, 0)),
            pl.BlockSpec((1, 8, 128, 24),
                         lambda i, r: (i, jnp.minimum(16 * r + 16, 31), 0, 0)),
            pl.BlockSpec((72, 128), lambda i, r: (0, 0)),
            pl.BlockSpec((1, 128), lambda i, r: (0, 0)),
        ],
        out_specs=pl.BlockSpec((1, 128, 128, 128), lambda i, r: (i, r, 0, 0)),
        out_shape=jax.ShapeDtypeStruct((B, 256, 128, 128), jnp.bfloat16),
    )(y24, y24, w72, b)


def _conv2_body(x_ref, w1_ref, w2_ref, b_ref, o_ref):
    x1c = x_ref[0].reshape(128, 2, 128, 128)
    zrow = jnp.zeros((1, 128, 128), jnp.bfloat16)
    even = x1c[:, 0]
    odd = x1c[:, 1]
    slabs = (
        jnp.concatenate([zrow, odd[0:127]], axis=0),
        even,
        odd,
    )
    acc = jnp.zeros((128 * 128, 64), dtype=jnp.float32)
    for dy in range(3):
        s = slabs[dy].reshape(128 * 128, 128)
        acc = acc + jnp.dot(s, w2_ref[dy], preferred_element_type=jnp.float32)
        t = jnp.dot(s, w1_ref[dy], preferred_element_type=jnp.float32)
        t = t.reshape(128, 128, 64)
        t = jnp.pad(t[:, :127, :], ((0, 0), (1, 0), (0, 0)))
        acc = acc + t.reshape(128 * 128, 64)
    y = jax.nn.relu(acc + b_ref[...])
    y = y.reshape(16, 8, 16, 8, 64)
    y = jnp.mean(y, axis=1)
    y = jnp.mean(y, axis=2)
    o_ref[0] = y


def _conv2_pool(x1c, wd1, wd2, b):
    B = x1c.shape[0]
    return pl.pallas_call(
        _conv2_body,
        grid=(B,),
        in_specs=[
            pl.BlockSpec((1, 256, 128, 128), lambda i: (i, 0, 0, 0)),
            pl.BlockSpec((3, 128, 64), lambda i: (0, 0, 0)),
            pl.BlockSpec((3, 128, 64), lambda i: (0, 0, 0)),
            pl.BlockSpec((1, 64), lambda i: (0, 0)),
        ],
        out_specs=pl.BlockSpec((1, 16, 16, 64), lambda i: (i, 0, 0, 0)),
        out_shape=jax.ShapeDtypeStruct((B, 16, 16, 64), jnp.float32),
    )(x1c, wd1, wd2, b)


def _mlp_body(pf_ref, bw1_ref, bb1_ref, bw2_ref, bb2_ref, bw3_ref, bb3_ref,
              sw_ref, sb_ref, big_ref, small_ref):
    f32 = jnp.float32
    pf = pf_ref[0].astype(jnp.bfloat16)
    big = jax.nn.relu(jnp.dot(pf, bw1_ref[...], preferred_element_type=f32)
                      + bb1_ref[...])
    big = jax.nn.relu(jnp.dot(big.astype(jnp.bfloat16), bw2_ref[...],
                              preferred_element_type=f32) + bb2_ref[...])
    big_ref[0] = (jnp.dot(big.astype(jnp.bfloat16), bw3_ref[...],
                          preferred_element_type=f32) + bb3_ref[...])
    small_ref[0] = (jnp.dot(pf, sw_ref[...], preferred_element_type=f32)
                    + sb_ref[...])


def _expert_mlps(pf, bw1, bb1, bw2, bb2, bw3, bb3, sw, sb):
    B = pf.shape[0]

    def rep(shape):
        nd = len(shape)
        return pl.BlockSpec(shape, lambda i, _n=nd: (0,) * _n)

    return pl.pallas_call(
        _mlp_body,
        grid=(B,),
        in_specs=[
            pl.BlockSpec((1, 256, 3072), lambda i: (i, 0, 0)),
            rep((3072, 512)), rep((1, 512)), rep((512, 512)), rep((1, 512)),
            rep((512, 128)), rep((1, 128)),
            rep((3072, 128)), rep((1, 128)),
        ],
        out_specs=[
            pl.BlockSpec((1, 256, 128), lambda i: (i, 0, 0)),
            pl.BlockSpec((1, 256, 128), lambda i: (i, 0, 0)),
        ],
        out_shape=[
            jax.ShapeDtypeStruct((B, 256, 128), jnp.float32),
            jax.ShapeDtypeStruct((B, 256, 128), jnp.float32),
        ],
    )(pf, bw1, bb1, bw2, bb2, bw3, bb3, sw, sb)


def _head_body(feat_ref, big_ref, small_ref, aw1_ref, ab1_ref, aw2_ref,
               ab2_ref, thr_ref, gw_ref, gb_ref, dw1_ref, db1_ref,
               cw_ref, cb_ref, rw_ref, rb_ref, cls_ref, reg_ref):
    f32 = jnp.float32
    feat = feat_ref[...].reshape(8 * 256, 64)
    big = big_ref[...]
    small = small_ref[...]

    h = jax.nn.relu(jnp.dot(feat, aw1_ref[...], preferred_element_type=f32)
                    + ab1_ref[...])
    s = jax.nn.sigmoid(jnp.dot(h, aw2_ref[...], preferred_element_type=f32)
                       + ab2_ref[...])
    s3 = s.reshape(8, 256, 1)
    mask = jax.nn.sigmoid(s3 - thr_ref[0, 0])
    combined = mask * big + (1.0 - mask) * small

    m = jnp.max(s3, axis=1, keepdims=True)
    e = jnp.exp(s3 - m)
    attn = e / jnp.sum(e, axis=1, keepdims=True)
    weighted = jnp.sum(combined * attn, axis=1)
    mean_pool = jnp.mean(combined, axis=1)
    gw = gw_ref[...]
    gfeat = (jnp.dot(weighted, gw[:128], preferred_element_type=f32)
             + jnp.dot(mean_pool, gw[128:], preferred_element_type=f32)
             + gb_ref[...])

    dw1 = dw1_ref[...]
    hc = jnp.dot(combined.reshape(8 * 256, 128), dw1[:128],
                 preferred_element_type=f32).reshape(8, 256, 256)
    hg = jnp.dot(gfeat, dw1[128:], preferred_element_type=f32)
    hd = jax.nn.relu(hc + hg[:, None, :] + db1_ref[...])
    hd2 = hd.reshape(8 * 256, 256)
    cls = jnp.dot(hd2, cw_ref[...], preferred_element_type=f32) + cb_ref[...]
    reg = jnp.dot(hd2, rw_ref[...], preferred_element_type=f32) + rb_ref[...]
    cls_ref[...] = cls.reshape(8, 256, 80)
    reg_ref[...] = reg.reshape(8, 256, 4)


def _head(feat, big, small, aw1, ab1, aw2, ab2, thr, gw, gb, dw1, db1,
          cw, cb, rw, rb):
    return pl.pallas_call(
        _head_body,
        out_shape=[
            jax.ShapeDtypeStruct((8, 256, 80), jnp.float32),
            jax.ShapeDtypeStruct((8, 256, 4), jnp.float32),
        ],
    )(feat, big, small, aw1, ab1, aw2, ab2, thr, gw, gb, dw1, db1,
      cw, cb, rw, rb)


def _conv1_w72(conv1_w):
    w = jnp.zeros((72, 128), jnp.float32)
    for A in range(2):
        for CB in range(-1, 2):
            for dr in range(2):
                for dc in range(2):
                    ky = 2 * A + dr - 1
                    if not 0 <= ky <= 2:
                        continue
                    base = A * 36 + (CB + 1) * 12 + (dr * 2 + dc) * 3
                    kx_e = 2 * CB + dc + 1
                    if 0 <= kx_e <= 2:
                        w = w.at[base:base + 3, 0:64].set(
                            conv1_w[:, :, ky, kx_e].T)
                    kx_o = 2 * CB + dc - 1
                    if 0 <= kx_o <= 2:
                        w = w.at[base:base + 3, 64:128].set(
                            conv1_w[:, :, ky, kx_o].T)
    return w


def _conv2_w(conv2_w):
    wt = conv2_w.transpose(2, 3, 1, 0)
    z = jnp.zeros((64, 64), jnp.float32)
    wd1 = jnp.stack([jnp.concatenate([z, wt[dy, 0]], axis=0)
                     for dy in range(3)])
    wd2 = jnp.stack([jnp.concatenate([wt[dy, 1], wt[dy, 2]], axis=0)
                     for dy in range(3)])
    return wd1, wd2


def kernel(images, patches, conv1_w, conv1_b, conv2_w, conv2_b, attn_w1,
           attn_b1, attn_w2, attn_b2, threshold, big_w1, big_b1, big_w2,
           big_b2, big_w3, big_b3, small_w, small_b, agg_w, agg_b, det_w1,
           det_b1, det_cls_w, det_cls_b, det_reg_w, det_reg_b):
    B = images.shape[0]
    N = 256

    y24 = images.astype(jnp.bfloat16).reshape(B, 3, 256, 2, 128, 2, 2)
    y24 = y24.transpose(0, 2, 4, 5, 3, 6, 1).reshape(B, 256, 128, 24)

    w72 = _conv1_w72(conv1_w).astype(jnp.bfloat16)
    b1 = jnp.concatenate([conv1_b, conv1_b]).reshape(1, 128)
    x1c = _conv1(y24, w72, b1)

    wd1, wd2 = _conv2_w(conv2_w)
    feat = _conv2_pool(x1c, wd1.astype(jnp.bfloat16), wd2.astype(jnp.bfloat16),
                       conv2_b.reshape(1, 64))
    feat = feat.reshape(B, N, 64)

    bf16 = jnp.bfloat16
    pf = patches.reshape(B, N, 3072)
    big, small = _expert_mlps(
        pf,
        big_w1.astype(bf16), big_b1.reshape(1, 512),
        big_w2.astype(bf16), big_b2.reshape(1, 512),
        big_w3.astype(bf16), big_b3.reshape(1, 128),
        small_w.astype(bf16), small_b.reshape(1, 128))

    cls_logits, reg_preds = _head(
        feat, big, small,
        attn_w1, attn_b1.reshape(1, 32), attn_w2, attn_b2.reshape(1, 1),
        threshold.reshape(1, 1),
        agg_w, agg_b.reshape(1, 256),
        det_w1, det_b1.reshape(1, 256),
        det_cls_w, det_cls_b.reshape(1, 80),
        det_reg_w, det_reg_b.reshape(1, 4))
    return (cls_logits, reg_preds)


# speedup vs baseline: 1.3601x; 1.0103x over previous
import jax
import jax.numpy as jnp
from jax.experimental import pallas as pl


def _conv1_body(y_ref, yh_ref, w_ref, b_ref, o_ref):
    y = y_ref[0]
    last = jnp.equal(pl.program_id(1), 1)
    hrow = jnp.where(last, jnp.zeros((1, 128, 24), jnp.bfloat16),
                     yh_ref[0, 0:1])
    ys = y[..., 12:24]
    hs = hrow[..., 12:24]
    prev = jnp.pad(ys, ((0, 0), (1, 0), (0, 0)))[:, :128, :]
    hprev = jnp.pad(hs, ((0, 0), (1, 0), (0, 0)))[:, :128, :]
    x36 = jnp.concatenate([prev, y], axis=-1)
    h36 = jnp.concatenate([hprev, hrow], axis=-1)
    xb = jnp.concatenate([x36[1:], h36], axis=0)
    a = x36.reshape(128 * 128, 36)
    an = xb.reshape(128 * 128, 36)
    out = (jnp.dot(a, w_ref[0:36], preferred_element_type=jnp.float32)
           + jnp.dot(an, w_ref[36:72], preferred_element_type=jnp.float32))
    out = jax.nn.relu(out + b_ref[...])
    o_ref[0] = out.astype(jnp.bfloat16).reshape(128, 128, 128)


def _conv1(y24, w72, b):
    B = y24.shape[0]
    return pl.pallas_call(
        _conv1_body,
        grid=(B, 2),
        in_specs=[
            pl.BlockSpec((1, 128, 128, 24), lambda i, r: (i, r, 0, 0)),
            pl.BlockSpec((1, 8, 128, 24),
                         lambda i, r: (i, jnp.minimum(16 * r + 16, 31), 0, 0)),
            pl.BlockSpec((72, 128), lambda i, r: (0, 0)),
            pl.BlockSpec((1, 128), lambda i, r: (0, 0)),
        ],
        out_specs=pl.BlockSpec((1, 128, 128, 128), lambda i, r: (i, r, 0, 0)),
        out_shape=jax.ShapeDtypeStruct((B, 256, 128, 128), jnp.bfloat16),
    )(y24, y24, w72, b)


def _conv2_body(x_ref, w1_ref, w2_ref, b_ref, o_ref):
    x1c = x_ref[0].reshape(128, 2, 128, 128)
    zrow = jnp.zeros((1, 128, 128), jnp.bfloat16)
    even = x1c[:, 0]
    odd = x1c[:, 1]
    up = jnp.concatenate([zrow, odd[0:127]], axis=0)
    xk = jnp.concatenate([up, even, odd], axis=-1)
    s = xk.reshape(128 * 128, 384)
    acc = jnp.dot(s, w2_ref[...], preferred_element_type=jnp.float32)
    t = jnp.dot(s, w1_ref[...], preferred_element_type=jnp.float32)
    t = t.reshape(128, 128, 64)
    t = jnp.pad(t[:, :127, :], ((0, 0), (1, 0), (0, 0)))
    acc = acc + t.reshape(128 * 128, 64)
    y = jax.nn.relu(acc + b_ref[...])
    y = y.reshape(16, 8, 16, 8, 64)
    y = jnp.mean(y, axis=1)
    y = jnp.mean(y, axis=2)
    o_ref[0] = y


def _conv2_pool(x1c, wd1, wd2, b):
    B = x1c.shape[0]
    return pl.pallas_call(
        _conv2_body,
        grid=(B,),
        in_specs=[
            pl.BlockSpec((1, 256, 128, 128), lambda i: (i, 0, 0, 0)),
            pl.BlockSpec((384, 64), lambda i: (0, 0)),
            pl.BlockSpec((384, 64), lambda i: (0, 0)),
            pl.BlockSpec((1, 64), lambda i: (0, 0)),
        ],
        out_specs=pl.BlockSpec((1, 16, 16, 64), lambda i: (i, 0, 0, 0)),
        out_shape=jax.ShapeDtypeStruct((B, 16, 16, 64), jnp.float32),
    )(x1c, wd1, wd2, b)


def _mlp_body(pf_ref, bw1_ref, bb1_ref, bw2_ref, bb2_ref, bw3_ref, bb3_ref,
              sw_ref, sb_ref, big_ref, small_ref):
    f32 = jnp.float32
    pf = pf_ref[0].astype(jnp.bfloat16)
    big = jax.nn.relu(jnp.dot(pf, bw1_ref[...], preferred_element_type=f32)
                      + bb1_ref[...])
    big = jax.nn.relu(jnp.dot(big.astype(jnp.bfloat16), bw2_ref[...],
                              preferred_element_type=f32) + bb2_ref[...])
    big_ref[0] = (jnp.dot(big.astype(jnp.bfloat16), bw3_ref[...],
                          preferred_element_type=f32) + bb3_ref[...])
    small_ref[0] = (jnp.dot(pf, sw_ref[...], preferred_element_type=f32)
                    + sb_ref[...])


def _expert_mlps(pf, bw1, bb1, bw2, bb2, bw3, bb3, sw, sb):
    B = pf.shape[0]

    def rep(shape):
        nd = len(shape)
        return pl.BlockSpec(shape, lambda i, _n=nd: (0,) * _n)

    return pl.pallas_call(
        _mlp_body,
        grid=(B,),
        in_specs=[
            pl.BlockSpec((1, 256, 3072), lambda i: (i, 0, 0)),
            rep((3072, 512)), rep((1, 512)), rep((512, 512)), rep((1, 512)),
            rep((512, 128)), rep((1, 128)),
            rep((3072, 128)), rep((1, 128)),
        ],
        out_specs=[
            pl.BlockSpec((1, 256, 128), lambda i: (i, 0, 0)),
            pl.BlockSpec((1, 256, 128), lambda i: (i, 0, 0)),
        ],
        out_shape=[
            jax.ShapeDtypeStruct((B, 256, 128), jnp.float32),
            jax.ShapeDtypeStruct((B, 256, 128), jnp.float32),
        ],
    )(pf, bw1, bb1, bw2, bb2, bw3, bb3, sw, sb)


def _head_body(feat_ref, big_ref, small_ref, aw1_ref, ab1_ref, aw2_ref,
               ab2_ref, thr_ref, gw_ref, gb_ref, dw1_ref, db1_ref,
               cw_ref, cb_ref, rw_ref, rb_ref, cls_ref, reg_ref):
    f32 = jnp.float32
    feat = feat_ref[...].reshape(8 * 256, 64)
    big = big_ref[...]
    small = small_ref[...]

    h = jax.nn.relu(jnp.dot(feat, aw1_ref[...], preferred_element_type=f32)
                    + ab1_ref[...])
    s = jax.nn.sigmoid(jnp.dot(h, aw2_ref[...], preferred_element_type=f32)
                       + ab2_ref[...])
    s3 = s.reshape(8, 256, 1)
    mask = jax.nn.sigmoid(s3 - thr_ref[0, 0])
    combined = mask * big + (1.0 - mask) * small

    m = jnp.max(s3, axis=1, keepdims=True)
    e = jnp.exp(s3 - m)
    attn = e / jnp.sum(e, axis=1, keepdims=True)
    weighted = jnp.sum(combined * attn, axis=1)
    mean_pool = jnp.mean(combined, axis=1)
    gw = gw_ref[...]
    gfeat = (jnp.dot(weighted, gw[:128], preferred_element_type=f32)
             + jnp.dot(mean_pool, gw[128:], preferred_element_type=f32)
             + gb_ref[...])

    dw1 = dw1_ref[...]
    hc = jnp.dot(combined.reshape(8 * 256, 128), dw1[:128],
                 preferred_element_type=f32).reshape(8, 256, 256)
    hg = jnp.dot(gfeat, dw1[128:], preferred_element_type=f32)
    hd = jax.nn.relu(hc + hg[:, None, :] + db1_ref[...])
    hd2 = hd.reshape(8 * 256, 256)
    cls = jnp.dot(hd2, cw_ref[...], preferred_element_type=f32) + cb_ref[...]
    reg = jnp.dot(hd2, rw_ref[...], preferred_element_type=f32) + rb_ref[...]
    cls_ref[...] = cls.reshape(8, 256, 80)
    reg_ref[...] = reg.reshape(8, 256, 4)


def _head(feat, big, small, aw1, ab1, aw2, ab2, thr, gw, gb, dw1, db1,
          cw, cb, rw, rb):
    return pl.pallas_call(
        _head_body,
        out_shape=[
            jax.ShapeDtypeStruct((8, 256, 80), jnp.float32),
            jax.ShapeDtypeStruct((8, 256, 4), jnp.float32),
        ],
    )(feat, big, small, aw1, ab1, aw2, ab2, thr, gw, gb, dw1, db1,
      cw, cb, rw, rb)


def _conv1_w72(conv1_w):
    w = jnp.zeros((72, 128), jnp.float32)
    for A in range(2):
        for CB in range(-1, 2):
            for dr in range(2):
                for dc in range(2):
                    ky = 2 * A + dr - 1
                    if not 0 <= ky <= 2:
                        continue
                    base = A * 36 + (CB + 1) * 12 + (dr * 2 + dc) * 3
                    kx_e = 2 * CB + dc + 1
                    if 0 <= kx_e <= 2:
                        w = w.at[base:base + 3, 0:64].set(
                            conv1_w[:, :, ky, kx_e].T)
                    kx_o = 2 * CB + dc - 1
                    if 0 <= kx_o <= 2:
                        w = w.at[base:base + 3, 64:128].set(
                            conv1_w[:, :, ky, kx_o].T)
    return w


def _conv2_w(conv2_w):
    wt = conv2_w.transpose(2, 3, 1, 0)
    z = jnp.zeros((64, 64), jnp.float32)
    wd1 = jnp.stack([jnp.concatenate([z, wt[dy, 0]], axis=0)
                     for dy in range(3)])
    wd2 = jnp.stack([jnp.concatenate([wt[dy, 1], wt[dy, 2]], axis=0)
                     for dy in range(3)])
    return wd1, wd2


def kernel(images, patches, conv1_w, conv1_b, conv2_w, conv2_b, attn_w1,
           attn_b1, attn_w2, attn_b2, threshold, big_w1, big_b1, big_w2,
           big_b2, big_w3, big_b3, small_w, small_b, agg_w, agg_b, det_w1,
           det_b1, det_cls_w, det_cls_b, det_reg_w, det_reg_b):
    B = images.shape[0]
    N = 256

    y24 = images.astype(jnp.bfloat16).reshape(B, 3, 256, 2, 128, 2, 2)
    y24 = y24.transpose(0, 2, 4, 5, 3, 6, 1).reshape(B, 256, 128, 24)

    w72 = _conv1_w72(conv1_w).astype(jnp.bfloat16)
    b1 = jnp.concatenate([conv1_b, conv1_b]).reshape(1, 128)
    x1c = _conv1(y24, w72, b1)

    wd1, wd2 = _conv2_w(conv2_w)
    feat = _conv2_pool(x1c, wd1.astype(jnp.bfloat16).reshape(384, 64),
                       wd2.astype(jnp.bfloat16).reshape(384, 64),
                       conv2_b.reshape(1, 64))
    feat = feat.reshape(B, N, 64)

    bf16 = jnp.bfloat16
    pf = patches.reshape(B, N, 3072)
    big, small = _expert_mlps(
        pf,
        big_w1.astype(bf16), big_b1.reshape(1, 512),
        big_w2.astype(bf16), big_b2.reshape(1, 512),
        big_w3.astype(bf16), big_b3.reshape(1, 128),
        small_w.astype(bf16), small_b.reshape(1, 128))

    cls_logits, reg_preds = _head(
        feat, big, small,
        attn_w1, attn_b1.reshape(1, 32), attn_w2, attn_b2.reshape(1, 1),
        threshold.reshape(1, 1),
        agg_w, agg_b.reshape(1, 256),
        det_w1, det_b1.reshape(1, 256),
        det_cls_w, det_cls_b.reshape(1, 80),
        det_reg_w, det_reg_b.reshape(1, 4))
    return (cls_logits, reg_preds)
